# speedup vs baseline: 6.4814x; 6.4814x over previous
import jax
import jax.numpy as jnp
from jax import lax
from jax.experimental import pallas as pl
from jax.experimental.pallas import tpu as pltpu
from jax.experimental.pallas import tpu_sc as plsc

N_U = 5000
N_I = 5000
N = N_U + N_I
NPAD = 10240
D = 128
E = 320000
NW = 32
EPT = NPAD
E_PAD = NW * EPT
NB = EPT // 128
L = 8192
LPT = L // NW
PAD_NODE = NPAD - 1
RB = 1024


def _sc_deg_body(col_hbm, degp_hbm, colv, onesv, zv, deg_sp):
    c = lax.axis_index("c")
    s = lax.axis_index("s")
    wid = c * 16 + s
    for k in range(8):
        onesv[pl.ds(k * 16, 16)] = jnp.full((16,), 1.0, jnp.float32)
    for k in range(40):
        zv[pl.ds(k * 16, 16)] = jnp.zeros((16,), jnp.float32)
    pltpu.sync_copy(col_hbm.at[wid], colv)
    pltpu.sync_copy(zv, deg_sp.at[pl.ds(s * 640, 640)])
    plsc.subcore_barrier()

    def body(j, carry):
        pltpu.sync_copy(onesv, deg_sp.at[colv.at[j]], add=True)
        return carry

    lax.fori_loop(0, NB, body, 0)
    plsc.subcore_barrier()
    pltpu.sync_copy(deg_sp.at[pl.ds(s * 640, 640)],
                    degp_hbm.at[c, pl.ds(s * 640, 640)])


def _sc_layer_body(u_hbm, row_hbm, col_hbm, part_hbm,
                   rowv, colv, rows, zb, sem, acc_sp):
    c = lax.axis_index("c")
    s = lax.axis_index("s")
    wid = c * 16 + s
    pltpu.sync_copy(row_hbm.at[wid], rowv)
    pltpu.sync_copy(col_hbm.at[wid], colv)

    def zbody(i, carry):
        for k in range(8):
            zb[i, pl.ds(k * 16, 16)] = jnp.zeros((16,), jnp.float32)
        return carry

    lax.fori_loop(0, 64, zbody, 0)
    for t in range(10):
        pltpu.sync_copy(zb, acc_sp.at[pl.ds(s * 640 + t * 64, 64)])
    plsc.subcore_barrier()

    def ebody(j, carry):
        pltpu.async_copy(u_hbm.at[rowv.at[j]], rows, sem).wait()
        pltpu.sync_copy(rows, acc_sp.at[colv.at[j]], add=True)
        return carry

    lax.fori_loop(0, NB, ebody, 0)
    plsc.subcore_barrier()
    for t in range(5):
        pltpu.sync_copy(acc_sp.at[pl.ds(s * 640 + t * 128, 128)],
                        part_hbm.at[c, pl.ds(s * 640 + t * 128, 128)])


def _sc_gather_body(out_hbm, ih_hbm, it_hbm, fh_hbm, ft_hbm, iv, rows, sem):
    c = lax.axis_index("c")
    s = lax.axis_index("s")
    wid = c * 16 + s
    pltpu.sync_copy(ih_hbm.at[wid], iv)
    for b in range(2):
        pltpu.async_copy(out_hbm.at[iv.at[b]], rows, sem).wait()
        pltpu.sync_copy(rows, fh_hbm.at[pl.ds(wid * LPT + b * 128, 128)])
    pltpu.sync_copy(it_hbm.at[wid], iv)
    for b in range(2):
        pltpu.async_copy(out_hbm.at[iv.at[b]], rows, sem).wait()
        pltpu.sync_copy(rows, ft_hbm.at[pl.ds(wid * LPT + b * 128, 128)])


def _tc_u0_body(degp_ref, x_ref, u0_ref):
    deg = degp_ref[0] + degp_ref[1]
    dinv = jnp.where(deg > 0, lax.rsqrt(deg), 0.0)
    u0_ref[...] = x_ref[...] * dinv


def _tc_u1_body(degp_ref, part_ref, u1_ref):
    agg = part_ref[0] + part_ref[1]
    deg = degp_ref[0] + degp_ref[1]
    w = jnp.where(deg > 0, 1.0 / deg, 0.0)
    u1_ref[...] = agg * w


def _tc_out_body(degp_ref, x_ref, p1_ref, p2_ref, out_ref):
    agg = p1_ref[0] + p1_ref[1] + p2_ref[0] + p2_ref[1]
    deg = degp_ref[0] + degp_ref[1]
    dinv = jnp.where(deg > 0, lax.rsqrt(deg), 0.0)
    out_ref[...] = (x_ref[...] + agg * dinv) * (1.0 / 3.0)


def _tc_dot_body(fh_ref, ft_ref, o_ref):
    o_ref[...] = jnp.sum(fh_ref[...] * ft_ref[...], axis=-1)


def kernel(x_user, x_item, edge_index, edge_label_index):
    x = jnp.concatenate([x_user, x_item], axis=0)
    x_pad = jnp.pad(x, ((0, NPAD - N), (0, 0)))
    pad = jnp.full((E_PAD - E,), PAD_NODE, jnp.int32)
    row_t = jnp.concatenate([edge_index[0], pad]).reshape(NW, NB, 128)
    col_t = jnp.concatenate([edge_index[1], pad]).reshape(NW, NB, 128)
    ih = edge_label_index[0].reshape(NW, 2, 128)
    it = (edge_label_index[1] + N_U).reshape(NW, 2, 128)

    mesh = plsc.VectorSubcoreMesh(core_axis_name="c", subcore_axis_name="s")

    deg_call = pl.kernel(
        _sc_deg_body,
        out_type=jax.ShapeDtypeStruct((2, NPAD), jnp.float32),
        mesh=mesh,
        scratch_types=[
            pltpu.VMEM((NB, 128), jnp.int32),
            pltpu.VMEM((128,), jnp.float32),
            pltpu.VMEM((640,), jnp.float32),
            pltpu.VMEM_SHARED((NPAD,), jnp.float32),
        ],
    )
    degp = deg_call(col_t)
    degp3 = degp.reshape(2, NPAD, 1)

    u0 = pl.pallas_call(
        _tc_u0_body,
        out_shape=jax.ShapeDtypeStruct((NPAD, D), jnp.float32),
        grid=(NPAD // RB,),
        in_specs=[
            pl.BlockSpec((2, RB, 1), lambda i: (0, i, 0)),
            pl.BlockSpec((RB, D), lambda i: (i, 0)),
        ],
        out_specs=pl.BlockSpec((RB, D), lambda i: (i, 0)),
    )(degp3, x_pad)

    layer_call = pl.kernel(
        _sc_layer_body,
        out_type=jax.ShapeDtypeStruct((2, NPAD, D), jnp.float32),
        mesh=mesh,
        scratch_types=[
            pltpu.VMEM((NB, 128), jnp.int32),
            pltpu.VMEM((NB, 128), jnp.int32),
            pltpu.VMEM((128, D), jnp.float32),
            pltpu.VMEM((64, D), jnp.float32),
            pltpu.SemaphoreType.DMA,
            pltpu.VMEM_SHARED((NPAD, D), jnp.float32),
        ],
    )
    part1 = layer_call(u0, row_t, col_t)

    u1 = pl.pallas_call(
        _tc_u1_body,
        out_shape=jax.ShapeDtypeStruct((NPAD, D), jnp.float32),
        grid=(NPAD // RB,),
        in_specs=[
            pl.BlockSpec((2, RB, 1), lambda i: (0, i, 0)),
            pl.BlockSpec((2, RB, D), lambda i: (0, i, 0)),
        ],
        out_specs=pl.BlockSpec((RB, D), lambda i: (i, 0)),
    )(degp3, part1)

    part2 = layer_call(u1, row_t, col_t)

    out = pl.pallas_call(
        _tc_out_body,
        out_shape=jax.ShapeDtypeStruct((NPAD, D), jnp.float32),
        grid=(NPAD // RB,),
        in_specs=[
            pl.BlockSpec((2, RB, 1), lambda i: (0, i, 0)),
            pl.BlockSpec((RB, D), lambda i: (i, 0)),
            pl.BlockSpec((2, RB, D), lambda i: (0, i, 0)),
            pl.BlockSpec((2, RB, D), lambda i: (0, i, 0)),
        ],
        out_specs=pl.BlockSpec((RB, D), lambda i: (i, 0)),
    )(degp3, x_pad, part1, part2)

    gather_call = pl.kernel(
        _sc_gather_body,
        out_type=(jax.ShapeDtypeStruct((L, D), jnp.float32),
                  jax.ShapeDtypeStruct((L, D), jnp.float32)),
        mesh=mesh,
        scratch_types=[
            pltpu.VMEM((2, 128), jnp.int32),
            pltpu.VMEM((128, D), jnp.float32),
            pltpu.SemaphoreType.DMA,
        ],
    )
    fh, ft = gather_call(out, ih, it)

    pred64 = pl.pallas_call(
        _tc_dot_body,
        out_shape=jax.ShapeDtypeStruct((L // 128, 128), jnp.float32),
        grid=(L // RB,),
        in_specs=[
            pl.BlockSpec((8, 128, D), lambda i: (i, 0, 0)),
            pl.BlockSpec((8, 128, D), lambda i: (i, 0, 0)),
        ],
        out_specs=pl.BlockSpec((8, 128), lambda i: (i, 0)),
    )(fh.reshape(L // 128, 128, D), ft.reshape(L // 128, 128, D))

    return pred64.reshape(L)


# speedup vs baseline: 7.2179x; 1.1136x over previous
import jax
import jax.numpy as jnp
from jax import lax
from jax.experimental import pallas as pl
from jax.experimental.pallas import tpu as pltpu
from jax.experimental.pallas import tpu_sc as plsc

N_U = 5000
N_I = 5000
N = N_U + N_I
NPAD = 10240
D = 128
E = 320000
NW = 32
EPT = NPAD
E_PAD = NW * EPT
NB = EPT // 128
L = 8192
LPT = L // NW
PAD_NODE = NPAD - 1
RB = 1024


def _sc_deg_body(col_hbm, degp_hbm, colv, onesv, zv, sem, deg_sp):
    c = lax.axis_index("c")
    s = lax.axis_index("s")
    wid = c * 16 + s
    for k in range(8):
        onesv[pl.ds(k * 16, 16)] = jnp.full((16,), 1.0, jnp.float32)
    for k in range(40):
        zv[pl.ds(k * 16, 16)] = jnp.zeros((16,), jnp.float32)
    pltpu.sync_copy(col_hbm.at[wid], colv)
    pltpu.sync_copy(zv, deg_sp.at[pl.ds(s * 640, 640)])
    plsc.subcore_barrier()

    def chunk(q, carry):
        def fire(j, cc):
            pltpu.async_copy(onesv, deg_sp.at[colv.at[q * 8 + j]], sem,
                             add=True)
            return cc
        lax.fori_loop(0, 8, fire, 0)

        def drain(j, cc):
            pltpu.make_async_copy(onesv, deg_sp.at[colv.at[q * 8 + j]],
                                  sem).wait()
            return cc
        lax.fori_loop(0, 8, drain, 0)
        return carry

    lax.fori_loop(0, NB // 8, chunk, 0)
    plsc.subcore_barrier()
    pltpu.sync_copy(deg_sp.at[pl.ds(s * 640, 640)],
                    degp_hbm.at[c, pl.ds(s * 640, 640)])


def _sc_layer_body(u_hbm, pk_hbm, part_hbm,
                   pkv, ri0, ci0, ri1, ci1, rows0, rows1, zb,
                   sg0, sg1, ss0, ss1, acc_sp):
    c = lax.axis_index("c")
    s = lax.axis_index("s")
    wid = c * 16 + s
    pltpu.sync_copy(pk_hbm.at[wid], pkv)

    def zbody(i, carry):
        for k in range(8):
            zb[i, pl.ds(k * 16, 16)] = jnp.zeros((16,), jnp.float32)
        return carry

    lax.fori_loop(0, 16, zbody, 0)
    for t in range(40):
        pltpu.async_copy(zb, acc_sp.at[pl.ds(s * 640 + t * 16, 16)], ss0)
    for t in range(40):
        pltpu.make_async_copy(zb, acc_sp.at[pl.ds(s * 640 + t * 16, 16)],
                              ss0).wait()
    plsc.subcore_barrier()

    def unpack(j, rib, cib):
        for k in range(8):
            v = pkv[j, pl.ds(k * 16, 16)]
            rib[pl.ds(k * 16, 16)] = lax.shift_right_logical(v, 14)
            cib[pl.ds(k * 16, 16)] = lax.bitwise_and(v, 16383)

    def _gather(rib, buf, sem):
        pltpu.async_copy(u_hbm.at[rib], buf, sem)

    def _gather_wait(rib, buf, sem):
        pltpu.make_async_copy(u_hbm.at[rib], buf, sem).wait()

    def _scat(cib, buf, sem):
        pltpu.async_copy(buf, acc_sp.at[cib], sem, add=True)

    def _scat_wait(cib, buf, sem):
        pltpu.make_async_copy(buf, acc_sp.at[cib], sem).wait()

    unpack(0, ri0, ci0)
    _gather(ri0, rows0, sg0)
    unpack(1, ri1, ci1)
    _gather(ri1, rows1, sg1)

    def pbody(jj, carry):
        j0 = 2 * jj
        j1 = j0 + 1
        _gather_wait(ri0, rows0, sg0)
        _scat(ci0, rows0, ss0)
        _gather_wait(ri1, rows1, sg1)
        _scat(ci1, rows1, ss1)
        _scat_wait(ci0, rows0, ss0)
        unpack(j0 + 2, ri0, ci0)
        _gather(ri0, rows0, sg0)
        _scat_wait(ci1, rows1, ss1)
        unpack(j1 + 2, ri1, ci1)
        _gather(ri1, rows1, sg1)
        return carry

    lax.fori_loop(0, (NB - 2) // 2, pbody, 0)
    _gather_wait(ri0, rows0, sg0)
    _scat(ci0, rows0, ss0)
    _gather_wait(ri1, rows1, sg1)
    _scat(ci1, rows1, ss1)
    _scat_wait(ci0, rows0, ss0)
    _scat_wait(ci1, rows1, ss1)
    plsc.subcore_barrier()
    for t in range(5):
        pltpu.async_copy(acc_sp.at[pl.ds(s * 640 + t * 128, 128)],
                         part_hbm.at[c, pl.ds(s * 640 + t * 128, 128)], sg0)
    for t in range(5):
        pltpu.make_async_copy(acc_sp.at[pl.ds(s * 640 + t * 128, 128)],
                              part_hbm.at[c, pl.ds(s * 640 + t * 128, 128)],
                              sg0).wait()


def _sc_gather_body(out_hbm, ih_hbm, it_hbm, fh_hbm, ft_hbm, iv, rows, sem):
    c = lax.axis_index("c")
    s = lax.axis_index("s")
    wid = c * 16 + s
    pltpu.sync_copy(ih_hbm.at[wid], iv)
    for b in range(2):
        pltpu.async_copy(out_hbm.at[iv.at[b]], rows, sem).wait()
        pltpu.sync_copy(rows, fh_hbm.at[pl.ds(wid * LPT + b * 128, 128)])
    pltpu.sync_copy(it_hbm.at[wid], iv)
    for b in range(2):
        pltpu.async_copy(out_hbm.at[iv.at[b]], rows, sem).wait()
        pltpu.sync_copy(rows, ft_hbm.at[pl.ds(wid * LPT + b * 128, 128)])


def _tc_u0_body(degp_ref, x_ref, u0_ref):
    deg = degp_ref[0] + degp_ref[1]
    dinv = jnp.where(deg > 0, lax.rsqrt(deg), 0.0)
    u0_ref[...] = x_ref[...] * dinv


def _tc_u1_body(degp_ref, part_ref, u1_ref):
    agg = part_ref[0] + part_ref[1]
    deg = degp_ref[0] + degp_ref[1]
    w = jnp.where(deg > 0, 1.0 / deg, 0.0)
    u1_ref[...] = agg * w


def _tc_out_body(degp_ref, x_ref, p1_ref, p2_ref, out_ref):
    agg = p1_ref[0] + p1_ref[1] + p2_ref[0] + p2_ref[1]
    deg = degp_ref[0] + degp_ref[1]
    dinv = jnp.where(deg > 0, lax.rsqrt(deg), 0.0)
    out_ref[...] = (x_ref[...] + agg * dinv) * (1.0 / 3.0)


def _tc_dot_body(fh_ref, ft_ref, o_ref):
    o_ref[...] = jnp.sum(fh_ref[...] * ft_ref[...], axis=-1)


def kernel(x_user, x_item, edge_index, edge_label_index):
    x = jnp.concatenate([x_user, x_item], axis=0)
    x_pad = jnp.pad(x, ((0, NPAD - N), (0, 0)))
    pad = jnp.full((E_PAD - E,), PAD_NODE, jnp.int32)
    row_p = jnp.concatenate([edge_index[0], pad])
    col_p = jnp.concatenate([edge_index[1], pad])
    col_t = col_p.reshape(NW, NB, 128)
    pk_t = ((row_p << 14) | col_p).reshape(NW, NB, 128)
    ih = edge_label_index[0].reshape(NW, 2, 128)
    it = (edge_label_index[1] + N_U).reshape(NW, 2, 128)

    mesh = plsc.VectorSubcoreMesh(core_axis_name="c", subcore_axis_name="s")

    deg_call = pl.kernel(
        _sc_deg_body,
        out_type=jax.ShapeDtypeStruct((2, NPAD), jnp.float32),
        mesh=mesh,
        scratch_types=[
            pltpu.VMEM((NB, 128), jnp.int32),
            pltpu.VMEM((128,), jnp.float32),
            pltpu.VMEM((640,), jnp.float32),
            pltpu.SemaphoreType.DMA,
            pltpu.VMEM_SHARED((NPAD,), jnp.float32),
        ],
    )
    degp = deg_call(col_t)
    degp3 = degp.reshape(2, NPAD, 1)

    u0 = pl.pallas_call(
        _tc_u0_body,
        out_shape=jax.ShapeDtypeStruct((NPAD, D), jnp.float32),
        grid=(NPAD // RB,),
        in_specs=[
            pl.BlockSpec((2, RB, 1), lambda i: (0, i, 0)),
            pl.BlockSpec((RB, D), lambda i: (i, 0)),
        ],
        out_specs=pl.BlockSpec((RB, D), lambda i: (i, 0)),
    )(degp3, x_pad)

    layer_call = pl.kernel(
        _sc_layer_body,
        out_type=jax.ShapeDtypeStruct((2, NPAD, D), jnp.float32),
        mesh=mesh,
        scratch_types=[
            pltpu.VMEM((NB, 128), jnp.int32),
            pltpu.VMEM((128,), jnp.int32),
            pltpu.VMEM((128,), jnp.int32),
            pltpu.VMEM((128,), jnp.int32),
            pltpu.VMEM((128,), jnp.int32),
            pltpu.VMEM((128, D), jnp.float32),
            pltpu.VMEM((128, D), jnp.float32),
            pltpu.VMEM((16, D), jnp.float32),
            pltpu.SemaphoreType.DMA,
            pltpu.SemaphoreType.DMA,
            pltpu.SemaphoreType.DMA,
            pltpu.SemaphoreType.DMA,
            pltpu.VMEM_SHARED((NPAD, D), jnp.float32),
        ],
    )
    part1 = layer_call(u0, pk_t)

    u1 = pl.pallas_call(
        _tc_u1_body,
        out_shape=jax.ShapeDtypeStruct((NPAD, D), jnp.float32),
        grid=(NPAD // RB,),
        in_specs=[
            pl.BlockSpec((2, RB, 1), lambda i: (0, i, 0)),
            pl.BlockSpec((2, RB, D), lambda i: (0, i, 0)),
        ],
        out_specs=pl.BlockSpec((RB, D), lambda i: (i, 0)),
    )(degp3, part1)

    part2 = layer_call(u1, pk_t)

    out = pl.pallas_call(
        _tc_out_body,
        out_shape=jax.ShapeDtypeStruct((NPAD, D), jnp.float32),
        grid=(NPAD // RB,),
        in_specs=[
            pl.BlockSpec((2, RB, 1), lambda i: (0, i, 0)),
            pl.BlockSpec((RB, D), lambda i: (i, 0)),
            pl.BlockSpec((2, RB, D), lambda i: (0, i, 0)),
            pl.BlockSpec((2, RB, D), lambda i: (0, i, 0)),
        ],
        out_specs=pl.BlockSpec((RB, D), lambda i: (i, 0)),
    )(degp3, x_pad, part1, part2)

    gather_call = pl.kernel(
        _sc_gather_body,
        out_type=(jax.ShapeDtypeStruct((L, D), jnp.float32),
                  jax.ShapeDtypeStruct((L, D), jnp.float32)),
        mesh=mesh,
        scratch_types=[
            pltpu.VMEM((2, 128), jnp.int32),
            pltpu.VMEM((128, D), jnp.float32),
            pltpu.SemaphoreType.DMA,
        ],
    )
    fh, ft = gather_call(out, ih, it)

    pred64 = pl.pallas_call(
        _tc_dot_body,
        out_shape=jax.ShapeDtypeStruct((L // 128, 128), jnp.float32),
        grid=(L // RB,),
        in_specs=[
            pl.BlockSpec((8, 128, D), lambda i: (i, 0, 0)),
            pl.BlockSpec((8, 128, D), lambda i: (i, 0, 0)),
        ],
        out_specs=pl.BlockSpec((8, 128), lambda i: (i, 0)),
    )(fh.reshape(L // 128, 128, D), ft.reshape(L // 128, 128, D))

    return pred64.reshape(L)


# speedup vs baseline: 7.2658x; 1.0066x over previous
import jax
import jax.numpy as jnp
from jax import lax
from jax.experimental import pallas as pl
from jax.experimental.pallas import tpu as pltpu
from jax.experimental.pallas import tpu_sc as plsc

N_U = 5000
N_I = 5000
N = N_U + N_I
NPAD = 10240
D = 128
E = 320000
NW = 32
EPT = NPAD
E_PAD = NW * EPT
NB = EPT // 128
L = 8192
LPT = L // NW
PAD_NODE = NPAD - 1
RB = 1024


def _sc_deg_body(col_hbm, degp_hbm, colv, onesv, zv, sem, deg_sp):
    c = lax.axis_index("c")
    s = lax.axis_index("s")
    wid = c * 16 + s
    for k in range(8):
        onesv[pl.ds(k * 16, 16)] = jnp.full((16,), 1.0, jnp.float32)
    for k in range(40):
        zv[pl.ds(k * 16, 16)] = jnp.zeros((16,), jnp.float32)
    pltpu.sync_copy(col_hbm.at[wid], colv)
    pltpu.sync_copy(zv, deg_sp.at[pl.ds(s * 640, 640)])
    plsc.subcore_barrier()

    def chunk(q, carry):
        def fire(j, cc):
            pltpu.async_copy(onesv, deg_sp.at[colv.at[q * 8 + j]], sem,
                             add=True)
            return cc
        lax.fori_loop(0, 8, fire, 0)

        def drain(j, cc):
            pltpu.make_async_copy(onesv, deg_sp.at[colv.at[q * 8 + j]],
                                  sem).wait()
            return cc
        lax.fori_loop(0, 8, drain, 0)
        return carry

    lax.fori_loop(0, NB // 8, chunk, 0)
    plsc.subcore_barrier()
    pltpu.sync_copy(deg_sp.at[pl.ds(s * 640, 640)],
                    degp_hbm.at[c, pl.ds(s * 640, 640)])


def _sc_layer_body(u_hbm, pk_hbm, part_hbm,
                   pkv, ri0, ci0, ri1, ci1, rows0, rows1, zb,
                   sg0, sg1, ss0, ss1, acc_sp):
    c = lax.axis_index("c")
    s = lax.axis_index("s")
    wid = c * 16 + s
    pltpu.sync_copy(pk_hbm.at[wid], pkv)

    def zbody(i, carry):
        for k in range(8):
            zb[i, pl.ds(k * 16, 16)] = jnp.zeros((16,), jnp.float32)
        return carry

    lax.fori_loop(0, 16, zbody, 0)
    for t in range(40):
        pltpu.async_copy(zb, acc_sp.at[pl.ds(s * 640 + t * 16, 16)], ss0)
    for t in range(40):
        pltpu.make_async_copy(zb, acc_sp.at[pl.ds(s * 640 + t * 16, 16)],
                              ss0).wait()
    plsc.subcore_barrier()

    def unpack(j, rib, cib):
        for k in range(8):
            v = pkv[j, pl.ds(k * 16, 16)]
            rib[pl.ds(k * 16, 16)] = lax.shift_right_logical(v, 14)
            cib[pl.ds(k * 16, 16)] = lax.bitwise_and(v, 16383)

    def _gather(rib, buf, sem):
        pltpu.async_copy(u_hbm.at[rib], buf, sem)

    def _gather_wait(rib, buf, sem):
        pltpu.make_async_copy(u_hbm.at[rib], buf, sem).wait()

    def _scat(cib, buf, sem):
        pltpu.async_copy(buf, acc_sp.at[cib], sem, add=True)

    def _scat_wait(cib, buf, sem):
        pltpu.make_async_copy(buf, acc_sp.at[cib], sem).wait()

    unpack(0, ri0, ci0)
    _gather(ri0, rows0, sg0)
    unpack(1, ri1, ci1)
    _gather(ri1, rows1, sg1)

    def pbody(jj, carry):
        j0 = 2 * jj
        j1 = j0 + 1
        _gather_wait(ri0, rows0, sg0)
        _scat(ci0, rows0, ss0)
        _gather_wait(ri1, rows1, sg1)
        _scat(ci1, rows1, ss1)
        _scat_wait(ci0, rows0, ss0)
        unpack(j0 + 2, ri0, ci0)
        _gather(ri0, rows0, sg0)
        _scat_wait(ci1, rows1, ss1)
        unpack(j1 + 2, ri1, ci1)
        _gather(ri1, rows1, sg1)
        return carry

    lax.fori_loop(0, (NB - 2) // 2, pbody, 0)
    _gather_wait(ri0, rows0, sg0)
    _scat(ci0, rows0, ss0)
    _gather_wait(ri1, rows1, sg1)
    _scat(ci1, rows1, ss1)
    _scat_wait(ci0, rows0, ss0)
    _scat_wait(ci1, rows1, ss1)
    plsc.subcore_barrier()
    for t in range(5):
        pltpu.async_copy(acc_sp.at[pl.ds(s * 640 + t * 128, 128)],
                         part_hbm.at[c, pl.ds(s * 640 + t * 128, 128)], sg0)
    for t in range(5):
        pltpu.make_async_copy(acc_sp.at[pl.ds(s * 640 + t * 128, 128)],
                              part_hbm.at[c, pl.ds(s * 640 + t * 128, 128)],
                              sg0).wait()


def _sc_gather_body(out_hbm, ih_hbm, it_hbm, fh_hbm, ft_hbm, iv, rows, sem):
    c = lax.axis_index("c")
    s = lax.axis_index("s")
    wid = c * 16 + s
    pltpu.sync_copy(ih_hbm.at[wid], iv)
    for b in range(2):
        pltpu.async_copy(out_hbm.at[iv.at[b]], rows, sem).wait()
        pltpu.sync_copy(rows, fh_hbm.at[pl.ds(wid * LPT + b * 128, 128)])
    pltpu.sync_copy(it_hbm.at[wid], iv)
    for b in range(2):
        pltpu.async_copy(out_hbm.at[iv.at[b]], rows, sem).wait()
        pltpu.sync_copy(rows, ft_hbm.at[pl.ds(wid * LPT + b * 128, 128)])


def _tc_u0_body(degp_ref, x_ref, u0_ref):
    deg = degp_ref[0] + degp_ref[1]
    dinv = jnp.where(deg > 0, lax.rsqrt(deg), 0.0)
    u0_ref[...] = x_ref[...] * dinv


def _tc_u1_body(degp_ref, part_ref, u1_ref):
    agg = part_ref[0] + part_ref[1]
    deg = degp_ref[0] + degp_ref[1]
    w = jnp.where(deg > 0, 1.0 / deg, 0.0)
    u1_ref[...] = agg * w


def _tc_out_body(degp_ref, x_ref, p1_ref, p2_ref, out_ref):
    agg = p1_ref[0] + p1_ref[1] + p2_ref[0] + p2_ref[1]
    deg = degp_ref[0] + degp_ref[1]
    dinv = jnp.where(deg > 0, lax.rsqrt(deg), 0.0)
    out_ref[...] = (x_ref[...] + agg * dinv) * (1.0 / 3.0)


def _tc_dot_body(fh_ref, ft_ref, o_ref):
    o_ref[...] = jnp.sum(fh_ref[...] * ft_ref[...], axis=-1)


def kernel(x_user, x_item, edge_index, edge_label_index):
    x = jnp.concatenate([x_user, x_item], axis=0)
    x_pad = jnp.pad(x, ((0, NPAD - N), (0, 0)))
    pad_row = jnp.full((E_PAD - E,), PAD_NODE, jnp.int32)
    pad_col = N + (jnp.arange(E_PAD - E, dtype=jnp.int32) % (NPAD - N))
    row_p = jnp.concatenate([edge_index[0], pad_row])
    col_p = jnp.concatenate([edge_index[1], pad_col])
    col_t = col_p.reshape(NW, NB, 128)
    pk_t = ((row_p << 14) | col_p).reshape(NW, NB, 128)
    ih = edge_label_index[0].reshape(NW, 2, 128)
    it = (edge_label_index[1] + N_U).reshape(NW, 2, 128)

    mesh = plsc.VectorSubcoreMesh(core_axis_name="c", subcore_axis_name="s")

    deg_call = pl.kernel(
        _sc_deg_body,
        out_type=jax.ShapeDtypeStruct((2, NPAD), jnp.float32),
        mesh=mesh,
        scratch_types=[
            pltpu.VMEM((NB, 128), jnp.int32),
            pltpu.VMEM((128,), jnp.float32),
            pltpu.VMEM((640,), jnp.float32),
            pltpu.SemaphoreType.DMA,
            pltpu.VMEM_SHARED((NPAD,), jnp.float32),
        ],
    )
    degp = deg_call(col_t)
    degp3 = degp.reshape(2, NPAD, 1)

    u0 = pl.pallas_call(
        _tc_u0_body,
        out_shape=jax.ShapeDtypeStruct((NPAD, D), jnp.float32),
        grid=(NPAD // RB,),
        in_specs=[
            pl.BlockSpec((2, RB, 1), lambda i: (0, i, 0)),
            pl.BlockSpec((RB, D), lambda i: (i, 0)),
        ],
        out_specs=pl.BlockSpec((RB, D), lambda i: (i, 0)),
    )(degp3, x_pad)

    layer_call = pl.kernel(
        _sc_layer_body,
        out_type=jax.ShapeDtypeStruct((2, NPAD, D), jnp.float32),
        mesh=mesh,
        scratch_types=[
            pltpu.VMEM((NB, 128), jnp.int32),
            pltpu.VMEM((128,), jnp.int32),
            pltpu.VMEM((128,), jnp.int32),
            pltpu.VMEM((128,), jnp.int32),
            pltpu.VMEM((128,), jnp.int32),
            pltpu.VMEM((128, D), jnp.float32),
            pltpu.VMEM((128, D), jnp.float32),
            pltpu.VMEM((16, D), jnp.float32),
            pltpu.SemaphoreType.DMA,
            pltpu.SemaphoreType.DMA,
            pltpu.SemaphoreType.DMA,
            pltpu.SemaphoreType.DMA,
            pltpu.VMEM_SHARED((NPAD, D), jnp.float32),
        ],
    )
    part1 = layer_call(u0, pk_t)

    u1 = pl.pallas_call(
        _tc_u1_body,
        out_shape=jax.ShapeDtypeStruct((NPAD, D), jnp.float32),
        grid=(NPAD // RB,),
        in_specs=[
            pl.BlockSpec((2, RB, 1), lambda i: (0, i, 0)),
            pl.BlockSpec((2, RB, D), lambda i: (0, i, 0)),
        ],
        out_specs=pl.BlockSpec((RB, D), lambda i: (i, 0)),
    )(degp3, part1)

    part2 = layer_call(u1, pk_t)

    out = pl.pallas_call(
        _tc_out_body,
        out_shape=jax.ShapeDtypeStruct((NPAD, D), jnp.float32),
        grid=(NPAD // RB,),
        in_specs=[
            pl.BlockSpec((2, RB, 1), lambda i: (0, i, 0)),
            pl.BlockSpec((RB, D), lambda i: (i, 0)),
            pl.BlockSpec((2, RB, D), lambda i: (0, i, 0)),
            pl.BlockSpec((2, RB, D), lambda i: (0, i, 0)),
        ],
        out_specs=pl.BlockSpec((RB, D), lambda i: (i, 0)),
    )(degp3, x_pad, part1, part2)

    gather_call = pl.kernel(
        _sc_gather_body,
        out_type=(jax.ShapeDtypeStruct((L, D), jnp.float32),
                  jax.ShapeDtypeStruct((L, D), jnp.float32)),
        mesh=mesh,
        scratch_types=[
            pltpu.VMEM((2, 128), jnp.int32),
            pltpu.VMEM((128, D), jnp.float32),
            pltpu.SemaphoreType.DMA,
        ],
    )
    fh, ft = gather_call(out, ih, it)

    pred64 = pl.pallas_call(
        _tc_dot_body,
        out_shape=jax.ShapeDtypeStruct((L // 128, 128), jnp.float32),
        grid=(L // RB,),
        in_specs=[
            pl.BlockSpec((8, 128, D), lambda i: (i, 0, 0)),
            pl.BlockSpec((8, 128, D), lambda i: (i, 0, 0)),
        ],
        out_specs=pl.BlockSpec((8, 128), lambda i: (i, 0)),
    )(fh.reshape(L // 128, 128, D), ft.reshape(L // 128, 128, D))

    return pred64.reshape(L)


# speedup vs baseline: 19.3593x; 2.6645x over previous
import jax
import jax.numpy as jnp
from jax import lax
from jax.experimental import pallas as pl
from jax.experimental.pallas import tpu as pltpu
from jax.experimental.pallas import tpu_sc as plsc

N_U = 5000
N_I = 5000
N = N_U + N_I
NPAD = 10240
D = 128
E = 320000
NW = 32
EPT = NPAD
E_PAD = NW * EPT
NB = EPT // 128
L = 8192
LPT = L // NW
PAD_NODE = NPAD - 1
RB = 1024


def _sc_deg_body(col_hbm, degp_hbm, colv, onesv, zv, sem, deg_sp):
    c = lax.axis_index("c")
    s = lax.axis_index("s")
    wid = c * 16 + s
    for k in range(8):
        onesv[pl.ds(k * 16, 16)] = jnp.full((16,), 1.0, jnp.float32)
    for k in range(40):
        zv[pl.ds(k * 16, 16)] = jnp.zeros((16,), jnp.float32)
    pltpu.sync_copy(col_hbm.at[wid], colv)
    pltpu.sync_copy(zv, deg_sp.at[pl.ds(s * 640, 640)])
    plsc.subcore_barrier()

    def chunk(q, carry):
        def fire(j, cc):
            pltpu.async_copy(onesv, deg_sp.at[colv.at[q * 8 + j]], sem,
                             add=True)
            return cc
        lax.fori_loop(0, 8, fire, 0)

        def drain(j, cc):
            pltpu.make_async_copy(onesv, deg_sp.at[colv.at[q * 8 + j]],
                                  sem).wait()
            return cc
        lax.fori_loop(0, 8, drain, 0)
        return carry

    lax.fori_loop(0, NB // 8, chunk, 0)
    plsc.subcore_barrier()
    pltpu.sync_copy(deg_sp.at[pl.ds(s * 640, 640)],
                    degp_hbm.at[c, pl.ds(s * 640, 640)])


def _sc_layer_body(u_hbm, pk_hbm, part_hbm,
                   pkv, ri0, ci0, ri1, ci1, rows0, rows1, zb,
                   sg0, sg1, ss0, ss1, acc_sp):
    c = lax.axis_index("c")
    s = lax.axis_index("s")
    wid = c * 16 + s
    pltpu.sync_copy(pk_hbm.at[wid], pkv)

    def zbody(i, carry):
        for k in range(8):
            zb[i, pl.ds(k * 16, 16)] = jnp.zeros((16,), jnp.float32)
        return carry

    lax.fori_loop(0, 16, zbody, 0)
    for t in range(40):
        pltpu.async_copy(zb, acc_sp.at[pl.ds(s * 640 + t * 16, 16)], ss0)
    for t in range(40):
        pltpu.make_async_copy(zb, acc_sp.at[pl.ds(s * 640 + t * 16, 16)],
                              ss0).wait()
    plsc.subcore_barrier()

    def unpack(j, rib, cib):
        for k in range(8):
            v = pkv[j, pl.ds(k * 16, 16)]
            rib[pl.ds(k * 16, 16)] = lax.shift_right_logical(v, 14)
            cib[pl.ds(k * 16, 16)] = lax.bitwise_and(v, 16383)

    def _gather(rib, buf, sem):
        pltpu.async_copy(u_hbm.at[rib], buf, sem)

    def _gather_wait(rib, buf, sem):
        pltpu.make_async_copy(u_hbm.at[rib], buf, sem).wait()

    def _scat(cib, buf, sem):
        pltpu.async_copy(buf, acc_sp.at[cib], sem, add=True)

    def _scat_wait(cib, buf, sem):
        pltpu.make_async_copy(buf, acc_sp.at[cib], sem).wait()

    unpack(0, ri0, ci0)
    _gather(ri0, rows0, sg0)
    unpack(1, ri1, ci1)
    _gather(ri1, rows1, sg1)

    def pbody(jj, carry):
        j0 = 2 * jj
        j1 = j0 + 1
        _gather_wait(ri0, rows0, sg0)
        _scat(ci0, rows0, ss0)
        _gather_wait(ri1, rows1, sg1)
        _scat(ci1, rows1, ss1)
        _scat_wait(ci0, rows0, ss0)
        unpack(j0 + 2, ri0, ci0)
        _gather(ri0, rows0, sg0)
        _scat_wait(ci1, rows1, ss1)
        unpack(j1 + 2, ri1, ci1)
        _gather(ri1, rows1, sg1)
        return carry

    lax.fori_loop(0, (NB - 2) // 2, pbody, 0)
    _gather_wait(ri0, rows0, sg0)
    _scat(ci0, rows0, ss0)
    _gather_wait(ri1, rows1, sg1)
    _scat(ci1, rows1, ss1)
    _scat_wait(ci0, rows0, ss0)
    _scat_wait(ci1, rows1, ss1)
    plsc.subcore_barrier()
    for t in range(5):
        pltpu.async_copy(acc_sp.at[pl.ds(s * 640 + t * 128, 128)],
                         part_hbm.at[c, pl.ds(s * 640 + t * 128, 128)], sg0)
    for t in range(5):
        pltpu.make_async_copy(acc_sp.at[pl.ds(s * 640 + t * 128, 128)],
                              part_hbm.at[c, pl.ds(s * 640 + t * 128, 128)],
                              sg0).wait()


def _sc_gather_body(out_hbm, ih_hbm, it_hbm, fh_hbm, ft_hbm, iv, rows, sem):
    c = lax.axis_index("c")
    s = lax.axis_index("s")
    wid = c * 16 + s
    pltpu.sync_copy(ih_hbm.at[wid], iv)
    for b in range(2):
        pltpu.async_copy(out_hbm.at[iv.at[b]], rows, sem).wait()
        pltpu.sync_copy(rows, fh_hbm.at[pl.ds(wid * LPT + b * 128, 128)])
    pltpu.sync_copy(it_hbm.at[wid], iv)
    for b in range(2):
        pltpu.async_copy(out_hbm.at[iv.at[b]], rows, sem).wait()
        pltpu.sync_copy(rows, ft_hbm.at[pl.ds(wid * LPT + b * 128, 128)])


def _tc_u0_body(degp_ref, x_ref, u0_ref):
    deg = degp_ref[0] + degp_ref[1]
    dinv = jnp.where(deg > 0, lax.rsqrt(deg), 0.0)
    u0_ref[...] = x_ref[...] * dinv


def _tc_u1_body(degp_ref, part_ref, u1_ref):
    agg = part_ref[0] + part_ref[1]
    deg = degp_ref[0] + degp_ref[1]
    w = jnp.where(deg > 0, 1.0 / deg, 0.0)
    u1_ref[...] = agg * w


def _tc_out_body(degp_ref, x_ref, p1_ref, p2_ref, out_ref):
    agg = p1_ref[0] + p1_ref[1] + p2_ref[0] + p2_ref[1]
    deg = degp_ref[0] + degp_ref[1]
    dinv = jnp.where(deg > 0, lax.rsqrt(deg), 0.0)
    out_ref[...] = (x_ref[...] + agg * dinv) * (1.0 / 3.0)


def _tc_dot_body(fh_ref, ft_ref, o_ref):
    o_ref[...] = jnp.sum(fh_ref[...] * ft_ref[...], axis=-1)


def kernel(x_user, x_item, edge_index, edge_label_index):
    x = jnp.concatenate([x_user, x_item], axis=0)
    x_pad = jnp.pad(x, ((0, NPAD - N), (0, 0)))
    pad_iota = jnp.arange(E_PAD - E, dtype=jnp.int32)
    pad_row = N + ((pad_iota + 97) % (NPAD - N))
    pad_col = N + (pad_iota % (NPAD - N))
    row_p = jnp.concatenate([edge_index[0], pad_row])
    col_p = jnp.concatenate([edge_index[1], pad_col])
    col_t = col_p.reshape(NW, NB, 128)
    pk_t = ((row_p << 14) | col_p).reshape(NW, NB, 128)
    ih = edge_label_index[0].reshape(NW, 2, 128)
    it = (edge_label_index[1] + N_U).reshape(NW, 2, 128)

    mesh = plsc.VectorSubcoreMesh(core_axis_name="c", subcore_axis_name="s")

    deg_call = pl.kernel(
        _sc_deg_body,
        out_type=jax.ShapeDtypeStruct((2, NPAD), jnp.float32),
        mesh=mesh,
        scratch_types=[
            pltpu.VMEM((NB, 128), jnp.int32),
            pltpu.VMEM((128,), jnp.float32),
            pltpu.VMEM((640,), jnp.float32),
            pltpu.SemaphoreType.DMA,
            pltpu.VMEM_SHARED((NPAD,), jnp.float32),
        ],
    )
    degp = deg_call(col_t)
    degp3 = degp.reshape(2, NPAD, 1)

    u0 = pl.pallas_call(
        _tc_u0_body,
        out_shape=jax.ShapeDtypeStruct((NPAD, D), jnp.float32),
        grid=(NPAD // RB,),
        in_specs=[
            pl.BlockSpec((2, RB, 1), lambda i: (0, i, 0)),
            pl.BlockSpec((RB, D), lambda i: (i, 0)),
        ],
        out_specs=pl.BlockSpec((RB, D), lambda i: (i, 0)),
    )(degp3, x_pad)

    layer_call = pl.kernel(
        _sc_layer_body,
        out_type=jax.ShapeDtypeStruct((2, NPAD, D), jnp.float32),
        mesh=mesh,
        scratch_types=[
            pltpu.VMEM((NB, 128), jnp.int32),
            pltpu.VMEM((128,), jnp.int32),
            pltpu.VMEM((128,), jnp.int32),
            pltpu.VMEM((128,), jnp.int32),
            pltpu.VMEM((128,), jnp.int32),
            pltpu.VMEM((128, D), jnp.float32),
            pltpu.VMEM((128, D), jnp.float32),
            pltpu.VMEM((16, D), jnp.float32),
            pltpu.SemaphoreType.DMA,
            pltpu.SemaphoreType.DMA,
            pltpu.SemaphoreType.DMA,
            pltpu.SemaphoreType.DMA,
            pltpu.VMEM_SHARED((NPAD, D), jnp.float32),
        ],
    )
    part1 = layer_call(u0, pk_t)

    u1 = pl.pallas_call(
        _tc_u1_body,
        out_shape=jax.ShapeDtypeStruct((NPAD, D), jnp.float32),
        grid=(NPAD // RB,),
        in_specs=[
            pl.BlockSpec((2, RB, 1), lambda i: (0, i, 0)),
            pl.BlockSpec((2, RB, D), lambda i: (0, i, 0)),
        ],
        out_specs=pl.BlockSpec((RB, D), lambda i: (i, 0)),
    )(degp3, part1)

    part2 = layer_call(u1, pk_t)

    out = pl.pallas_call(
        _tc_out_body,
        out_shape=jax.ShapeDtypeStruct((NPAD, D), jnp.float32),
        grid=(NPAD // RB,),
        in_specs=[
            pl.BlockSpec((2, RB, 1), lambda i: (0, i, 0)),
            pl.BlockSpec((RB, D), lambda i: (i, 0)),
            pl.BlockSpec((2, RB, D), lambda i: (0, i, 0)),
            pl.BlockSpec((2, RB, D), lambda i: (0, i, 0)),
        ],
        out_specs=pl.BlockSpec((RB, D), lambda i: (i, 0)),
    )(degp3, x_pad, part1, part2)

    gather_call = pl.kernel(
        _sc_gather_body,
        out_type=(jax.ShapeDtypeStruct((L, D), jnp.float32),
                  jax.ShapeDtypeStruct((L, D), jnp.float32)),
        mesh=mesh,
        scratch_types=[
            pltpu.VMEM((2, 128), jnp.int32),
            pltpu.VMEM((128, D), jnp.float32),
            pltpu.SemaphoreType.DMA,
        ],
    )
    fh, ft = gather_call(out, ih, it)

    pred64 = pl.pallas_call(
        _tc_dot_body,
        out_shape=jax.ShapeDtypeStruct((L // 128, 128), jnp.float32),
        grid=(L // RB,),
        in_specs=[
            pl.BlockSpec((8, 128, D), lambda i: (i, 0, 0)),
            pl.BlockSpec((8, 128, D), lambda i: (i, 0, 0)),
        ],
        out_specs=pl.BlockSpec((8, 128), lambda i: (i, 0)),
    )(fh.reshape(L // 128, 128, D), ft.reshape(L // 128, 128, D))

    return pred64.reshape(L)


# speedup vs baseline: 22.4935x; 1.1619x over previous
import jax
import jax.numpy as jnp
from jax import lax
from jax.experimental import pallas as pl
from jax.experimental.pallas import tpu as pltpu
from jax.experimental.pallas import tpu_sc as plsc

N_U = 5000
N_I = 5000
N = N_U + N_I
NPAD = 10240
D = 128
E = 320000
NW = 32
EB = 80
NB = 126
EPT = EB * NB
E_PAD = NW * EPT
L = 8192
LPT = L // NW
PAD_NODE = NPAD - 1
RB = 2048


def _sc_deg_body(col_hbm, degp_hbm, colv, onesv, zv, sem, deg_sp):
    c = lax.axis_index("c")
    s = lax.axis_index("s")
    wid = c * 16 + s
    for k in range(EB // 16):
        onesv[pl.ds(k * 16, 16)] = jnp.full((16,), 1.0, jnp.float32)
    for k in range(40):
        zv[pl.ds(k * 16, 16)] = jnp.zeros((16,), jnp.float32)
    pltpu.sync_copy(col_hbm.at[wid], colv)
    pltpu.sync_copy(zv, deg_sp.at[pl.ds(s * 640, 640)])
    plsc.subcore_barrier()

    def chunk(q, carry):
        def fire(j, cc):
            pltpu.async_copy(onesv, deg_sp.at[colv.at[q * 6 + j]], sem,
                             add=True)
            return cc
        lax.fori_loop(0, 6, fire, 0)

        def drain(j, cc):
            pltpu.make_async_copy(onesv, deg_sp.at[colv.at[q * 6 + j]],
                                  sem).wait()
            return cc
        lax.fori_loop(0, 6, drain, 0)
        return carry

    lax.fori_loop(0, NB // 6, chunk, 0)
    plsc.subcore_barrier()
    pltpu.sync_copy(deg_sp.at[pl.ds(s * 640, 640)],
                    degp_hbm.at[c, pl.ds(s * 640, 640)])


def _sc_layer_body(u_hbm, pk_hbm, part_hbm,
                   pkv, ri0, ci0, ri1, ci1, ri2, ci2,
                   rows0, rows1, rows2, zb,
                   sg0, sg1, sg2, ss0, ss1, ss2, acc_sp):
    c = lax.axis_index("c")
    s = lax.axis_index("s")
    wid = c * 16 + s
    pltpu.sync_copy(pk_hbm.at[wid], pkv)

    def zbody(i, carry):
        for k in range(8):
            zb[i, pl.ds(k * 16, 16)] = jnp.zeros((16,), jnp.float32)
        return carry

    lax.fori_loop(0, 8, zbody, 0)

    def zchunk(q, carry):
        for t in range(8):
            pltpu.async_copy(
                zb, acc_sp.at[pl.ds(s * 640 + (q * 8 + t) * 8, 8)], ss0)
        for t in range(8):
            pltpu.make_async_copy(
                zb, acc_sp.at[pl.ds(s * 640 + (q * 8 + t) * 8, 8)],
                ss0).wait()
        return carry

    lax.fori_loop(0, 10, zchunk, 0)
    plsc.subcore_barrier()

    def unpack(j, rib, cib):
        for k in range(EB // 16):
            v = pkv[j, pl.ds(k * 16, 16)]
            rib[pl.ds(k * 16, 16)] = lax.shift_right_logical(v, 14)
            cib[pl.ds(k * 16, 16)] = lax.bitwise_and(v, 16383)

    def _gather(rib, buf, sem):
        pltpu.async_copy(u_hbm.at[rib], buf, sem)

    def _gather_wait(rib, buf, sem):
        pltpu.make_async_copy(u_hbm.at[rib], buf, sem).wait()

    def _scat(cib, buf, sem):
        pltpu.async_copy(buf, acc_sp.at[cib], sem, add=True)

    def _scat_wait(cib, buf, sem):
        pltpu.make_async_copy(buf, acc_sp.at[cib], sem).wait()

    bufs = ((ri0, ci0, rows0, sg0, ss0),
            (ri1, ci1, rows1, sg1, ss1),
            (ri2, ci2, rows2, sg2, ss2))

    for p in range(3):
        rib, cib, buf, sg, ss = bufs[p]
        unpack(p, rib, cib)
        _gather(rib, buf, sg)

    def pbody(jj, carry):
        j0 = 3 * jj
        for p in range(3):
            rib, cib, buf, sg, ss = bufs[p]
            _gather_wait(rib, buf, sg)
            _scat(cib, buf, ss)
        for p in range(3):
            rib, cib, buf, sg, ss = bufs[p]
            _scat_wait(cib, buf, ss)
            unpack(j0 + p + 3, rib, cib)
            _gather(rib, buf, sg)
        return carry

    lax.fori_loop(0, NB // 3 - 1, pbody, 0)
    for p in range(3):
        rib, cib, buf, sg, ss = bufs[p]
        _gather_wait(rib, buf, sg)
        _scat(cib, buf, ss)
    for p in range(3):
        rib, cib, buf, sg, ss = bufs[p]
        _scat_wait(cib, buf, ss)
    plsc.subcore_barrier()
    for t in range(5):
        pltpu.async_copy(acc_sp.at[pl.ds(s * 640 + t * 128, 128)],
                         part_hbm.at[c, pl.ds(s * 640 + t * 128, 128)], sg0)
    for t in range(5):
        pltpu.make_async_copy(acc_sp.at[pl.ds(s * 640 + t * 128, 128)],
                              part_hbm.at[c, pl.ds(s * 640 + t * 128, 128)],
                              sg0).wait()


def _sc_cls_body(out_hbm, ih_hbm, it_hbm, fo_hbm,
                 ihv, itv, rh0, rt0, rh1, rt1, foldv,
                 s0, s1, s2, s3):
    c = lax.axis_index("c")
    s = lax.axis_index("s")
    wid = c * 16 + s
    pltpu.sync_copy(ih_hbm.at[wid], ihv)
    pltpu.sync_copy(it_hbm.at[wid], itv)
    pltpu.async_copy(out_hbm.at[ihv.at[0]], rh0, s0)
    pltpu.async_copy(out_hbm.at[itv.at[0]], rt0, s1)
    pltpu.async_copy(out_hbm.at[ihv.at[1]], rh1, s2)
    pltpu.async_copy(out_hbm.at[itv.at[1]], rt1, s3)

    for b, (rh, rt, sa, sb) in enumerate(((rh0, rt0, s0, s1),
                                          (rh1, rt1, s2, s3))):
        pltpu.make_async_copy(out_hbm.at[ihv.at[b]], rh, sa).wait()
        pltpu.make_async_copy(out_hbm.at[itv.at[b]], rt, sb).wait()

        def pair(p, carry):
            acc = rh[p, pl.ds(0, 16)] * rt[p, pl.ds(0, 16)]
            for k in range(1, 8):
                acc = acc + (rh[p, pl.ds(k * 16, 16)] *
                             rt[p, pl.ds(k * 16, 16)])
            foldv[p, pl.ds(0, 16)] = acc
            return carry

        lax.fori_loop(0, 128, pair, 0)
        pltpu.sync_copy(foldv, fo_hbm.at[pl.ds(wid * LPT + b * 128, 128)])


def _tc_u0_body(degp_ref, x_ref, u0_ref):
    deg = degp_ref[0] + degp_ref[1]
    dinv = jnp.where(deg > 0, lax.rsqrt(deg), 0.0)
    u0_ref[...] = x_ref[...] * dinv


def _tc_u1_body(degp_ref, part_ref, u1_ref):
    agg = part_ref[0] + part_ref[1]
    deg = degp_ref[0] + degp_ref[1]
    w = jnp.where(deg > 0, 1.0 / deg, 0.0)
    u1_ref[...] = agg * w


def _tc_out_body(degp_ref, x_ref, p1_ref, p2_ref, out_ref):
    agg = p1_ref[0] + p1_ref[1] + p2_ref[0] + p2_ref[1]
    deg = degp_ref[0] + degp_ref[1]
    dinv = jnp.where(deg > 0, lax.rsqrt(deg), 0.0)
    out_ref[...] = (x_ref[...] + agg * dinv) * (1.0 / 3.0)


def _tc_dot_body(fo_ref, o_ref):
    o_ref[...] = jnp.sum(fo_ref[...], axis=-1)


def kernel(x_user, x_item, edge_index, edge_label_index):
    x = jnp.concatenate([x_user, x_item], axis=0)
    x_pad = jnp.pad(x, ((0, NPAD - N), (0, 0)))
    pad_iota = jnp.arange(E_PAD - E, dtype=jnp.int32)
    pad_row = N + ((pad_iota + 97) % (NPAD - N))
    pad_col = N + (pad_iota % (NPAD - N))
    row_p = jnp.concatenate([edge_index[0], pad_row])
    col_p = jnp.concatenate([edge_index[1], pad_col])
    col_t = col_p.reshape(NW, NB, EB)
    pk_t = ((row_p << 14) | col_p).reshape(NW, NB, EB)
    ih = edge_label_index[0].reshape(NW, 2, 128)
    it = (edge_label_index[1] + N_U).reshape(NW, 2, 128)

    mesh = plsc.VectorSubcoreMesh(core_axis_name="c", subcore_axis_name="s")

    deg_call = pl.kernel(
        _sc_deg_body,
        out_type=jax.ShapeDtypeStruct((2, NPAD), jnp.float32),
        mesh=mesh,
        scratch_types=[
            pltpu.VMEM((NB, EB), jnp.int32),
            pltpu.VMEM((EB,), jnp.float32),
            pltpu.VMEM((640,), jnp.float32),
            pltpu.SemaphoreType.DMA,
            pltpu.VMEM_SHARED((NPAD,), jnp.float32),
        ],
    )
    degp = deg_call(col_t)
    degp3 = degp.reshape(2, NPAD, 1)

    u0 = pl.pallas_call(
        _tc_u0_body,
        out_shape=jax.ShapeDtypeStruct((NPAD, D), jnp.float32),
        grid=(NPAD // RB,),
        in_specs=[
            pl.BlockSpec((2, RB, 1), lambda i: (0, i, 0)),
            pl.BlockSpec((RB, D), lambda i: (i, 0)),
        ],
        out_specs=pl.BlockSpec((RB, D), lambda i: (i, 0)),
    )(degp3, x_pad)

    layer_call = pl.kernel(
        _sc_layer_body,
        out_type=jax.ShapeDtypeStruct((2, NPAD, D), jnp.float32),
        mesh=mesh,
        scratch_types=[
            pltpu.VMEM((NB, EB), jnp.int32),
            pltpu.VMEM((EB,), jnp.int32),
            pltpu.VMEM((EB,), jnp.int32),
            pltpu.VMEM((EB,), jnp.int32),
            pltpu.VMEM((EB,), jnp.int32),
            pltpu.VMEM((EB,), jnp.int32),
            pltpu.VMEM((EB,), jnp.int32),
            pltpu.VMEM((EB, D), jnp.float32),
            pltpu.VMEM((EB, D), jnp.float32),
            pltpu.VMEM((EB, D), jnp.float32),
            pltpu.VMEM((8, D), jnp.float32),
            pltpu.SemaphoreType.DMA,
            pltpu.SemaphoreType.DMA,
            pltpu.SemaphoreType.DMA,
            pltpu.SemaphoreType.DMA,
            pltpu.SemaphoreType.DMA,
            pltpu.SemaphoreType.DMA,
            pltpu.VMEM_SHARED((NPAD, D), jnp.float32),
        ],
    )
    part1 = layer_call(u0, pk_t)

    u1 = pl.pallas_call(
        _tc_u1_body,
        out_shape=jax.ShapeDtypeStruct((NPAD, D), jnp.float32),
        grid=(NPAD // RB,),
        in_specs=[
            pl.BlockSpec((2, RB, 1), lambda i: (0, i, 0)),
            pl.BlockSpec((2, RB, D), lambda i: (0, i, 0)),
        ],
        out_specs=pl.BlockSpec((RB, D), lambda i: (i, 0)),
    )(degp3, part1)

    part2 = layer_call(u1, pk_t)

    out = pl.pallas_call(
        _tc_out_body,
        out_shape=jax.ShapeDtypeStruct((NPAD, D), jnp.float32),
        grid=(NPAD // RB,),
        in_specs=[
            pl.BlockSpec((2, RB, 1), lambda i: (0, i, 0)),
            pl.BlockSpec((RB, D), lambda i: (i, 0)),
            pl.BlockSpec((2, RB, D), lambda i: (0, i, 0)),
            pl.BlockSpec((2, RB, D), lambda i: (0, i, 0)),
        ],
        out_specs=pl.BlockSpec((RB, D), lambda i: (i, 0)),
    )(degp3, x_pad, part1, part2)

    cls_call = pl.kernel(
        _sc_cls_body,
        out_type=jax.ShapeDtypeStruct((L, 16), jnp.float32),
        mesh=mesh,
        scratch_types=[
            pltpu.VMEM((2, 128), jnp.int32),
            pltpu.VMEM((2, 128), jnp.int32),
            pltpu.VMEM((128, D), jnp.float32),
            pltpu.VMEM((128, D), jnp.float32),
            pltpu.VMEM((128, D), jnp.float32),
            pltpu.VMEM((128, D), jnp.float32),
            pltpu.VMEM((128, 16), jnp.float32),
            pltpu.SemaphoreType.DMA,
            pltpu.SemaphoreType.DMA,
            pltpu.SemaphoreType.DMA,
            pltpu.SemaphoreType.DMA,
        ],
    )
    fo = cls_call(out, ih, it)

    pred64 = pl.pallas_call(
        _tc_dot_body,
        out_shape=jax.ShapeDtypeStruct((L // 128, 128), jnp.float32),
        grid=(L // RB,),
        in_specs=[
            pl.BlockSpec((RB // 128, 128, 16), lambda i: (i, 0, 0)),
        ],
        out_specs=pl.BlockSpec((RB // 128, 128), lambda i: (i, 0)),
    )(fo.reshape(L // 128, 128, 16))

    return pred64.reshape(L)
